# baseline (speedup 1.0000x reference)
import functools

import jax
import jax.numpy as jnp
from jax import lax
from jax.experimental import pallas as pl
from jax.experimental.pallas import tpu as pltpu
from jax.experimental.pallas import tpu_sc as plsc

NWORDS = 100000
EMB = 16
HID = 128
NHIST = 20
B = 1024

_TOTAL = B * NHIST
_NW = 32
_CHUNK = 128
_PER_W = _TOTAL // _NW
_NCH = _PER_W // _CHUNK
_BN = 5120
_NT = -(-NWORDS // _BN)
_TAIL = NWORDS - (_NT - 1) * _BN
_NBUF = 4


def _make_sc_gather():
    mesh = plsc.VectorSubcoreMesh(core_axis_name="c", subcore_axis_name="s")

    @functools.partial(
        pl.kernel,
        mesh=mesh,
        out_type=jax.ShapeDtypeStruct((_NW, _NCH, _CHUNK, EMB), jnp.float32),
        scratch_types=[
            pltpu.VMEM((_NCH, _CHUNK), jnp.int32),
            pltpu.VMEM((_NCH, _CHUNK, EMB), jnp.float32),
            pltpu.SemaphoreType.DMA,
        ],
        compiler_params=pltpu.CompilerParams(use_tc_tiling_on_sc=False),
    )
    def sc_gather(idx_hbm, table_hbm, out_hbm, idx_v, rows_v, sem):
        wid = lax.axis_index("s") * 2 + lax.axis_index("c")
        pltpu.sync_copy(idx_hbm.at[wid], idx_v)
        copies = [
            pltpu.async_copy(table_hbm.at[idx_v.at[j]], rows_v.at[j], sem)
            for j in range(_NCH)
        ]
        for c in copies:
            c.wait()
        pltpu.sync_copy(rows_v, out_hbm.at[wid])

    return sc_gather


_sc_gather_cache = []


def _sc_gather(idx, table):
    if not _sc_gather_cache:
        _sc_gather_cache.append(_make_sc_gather())
    return _sc_gather_cache[0](idx, table)


def _mlp_body(feat_ref, w1_ref, b1_ref, w2_ref, b2_ref, out_ref, h_ref):
    @pl.when(pl.program_id(0) == 0)
    def _():
        h = lax.dot_general(
            feat_ref[...], w1_ref[...],
            (((1,), (1,)), ((), ())),
            preferred_element_type=jnp.float32,
        )
        h_ref[...] = h + b1_ref[...]

    out_ref[...] = lax.dot_general(
        w2_ref[...], h_ref[...],
        (((1,), (1,)), ((), ())),
        preferred_element_type=jnp.float32,
    ) + b2_ref[...]


def _mlp(feat, W1, b1, W2, b2):
    outT = pl.pallas_call(
        _mlp_body,
        grid=(_NT,),
        in_specs=[
            pl.BlockSpec((B, NHIST * EMB), lambda i: (0, 0)),
            pl.BlockSpec((HID, NHIST * EMB), lambda i: (0, 0)),
            pl.BlockSpec((1, HID), lambda i: (0, 0)),
            pl.BlockSpec((_BN, HID), lambda i: (i, 0)),
            pl.BlockSpec((_BN, 1), lambda i: (i, 0)),
        ],
        out_specs=pl.BlockSpec((_BN, B), lambda i: (i, 0)),
        out_shape=jax.ShapeDtypeStruct((NWORDS, B), jnp.float32),
        scratch_shapes=[pltpu.VMEM((B, HID), jnp.float32)],
        compiler_params=pltpu.CompilerParams(
            vmem_limit_bytes=100 * 1024 * 1024,
        ),
    )(feat, W1, b1.reshape(1, HID), W2, b2.reshape(NWORDS, 1))
    return outT.T


def kernel(x, emb_table, W1, b1, W2, b2):
    idx = x.astype(jnp.int32).reshape(_NW, _NCH, _CHUNK)
    rows = _sc_gather(idx, emb_table)
    feat = rows.reshape(B, NHIST * EMB) * jnp.float32(1.0000001)
    return _mlp(feat, W1, b1, W2, b2)


# speedup vs baseline: 1.0879x; 1.0879x over previous
import functools

import jax
import jax.numpy as jnp
from jax import lax
from jax.experimental import pallas as pl
from jax.experimental.pallas import tpu as pltpu
from jax.experimental.pallas import tpu_sc as plsc

NWORDS = 100000
EMB = 16
HID = 128
NHIST = 20
B = 1024

_TOTAL = B * NHIST
_NW = 32
_CHUNK = 128
_PER_W = _TOTAL // _NW
_NCH = _PER_W // _CHUNK
_BN = 5120
_NT = -(-NWORDS // _BN)
_TAIL = NWORDS - (_NT - 1) * _BN
_NBUF = 4


def _make_sc_gather():
    mesh = plsc.VectorSubcoreMesh(core_axis_name="c", subcore_axis_name="s")

    @functools.partial(
        pl.kernel,
        mesh=mesh,
        out_type=jax.ShapeDtypeStruct((_NW, _NCH, _CHUNK, EMB), jnp.float32),
        scratch_types=[
            pltpu.VMEM((_NCH, _CHUNK), jnp.int32),
            pltpu.VMEM((_NCH, _CHUNK, EMB), jnp.float32),
            pltpu.SemaphoreType.DMA,
        ],
        compiler_params=pltpu.CompilerParams(use_tc_tiling_on_sc=False),
    )
    def sc_gather(idx_hbm, table_hbm, out_hbm, idx_v, rows_v, sem):
        wid = lax.axis_index("s") * 2 + lax.axis_index("c")
        pltpu.sync_copy(idx_hbm.at[wid], idx_v)
        copies = [
            pltpu.async_copy(table_hbm.at[idx_v.at[j]], rows_v.at[j], sem)
            for j in range(_NCH)
        ]
        for c in copies:
            c.wait()
        pltpu.sync_copy(rows_v, out_hbm.at[wid])

    return sc_gather


_sc_gather_cache = []


def _sc_gather(idx, table):
    if not _sc_gather_cache:
        _sc_gather_cache.append(_make_sc_gather())
    return _sc_gather_cache[0](idx, table)


def _mlp_body(feat_ref, w1_ref, b1_ref, w2_ref, b2_ref, out_ref, h_ref):
    @pl.when(pl.program_id(0) == 0)
    def _():
        h = lax.dot_general(
            feat_ref[...], w1_ref[...],
            (((1,), (1,)), ((), ())),
            preferred_element_type=jnp.float32,
        )
        h_ref[...] = h + b1_ref[...]

    out_ref[...] = lax.dot_general(
        w2_ref[...], h_ref[...],
        (((1,), (1,)), ((), ())),
        preferred_element_type=jnp.float32,
    ) + b2_ref[...]


def _mlp(feat, W1, b1, W2, b2):
    outT = pl.pallas_call(
        _mlp_body,
        grid=(_NT,),
        in_specs=[
            pl.BlockSpec((B, NHIST * EMB), lambda i: (0, 0)),
            pl.BlockSpec((HID, NHIST * EMB), lambda i: (0, 0)),
            pl.BlockSpec((1, HID), lambda i: (0, 0)),
            pl.BlockSpec((_BN, HID), lambda i: (i, 0)),
            pl.BlockSpec((_BN, 1), lambda i: (i, 0)),
        ],
        out_specs=pl.BlockSpec((_BN, B), lambda i: (i, 0)),
        out_shape=jax.ShapeDtypeStruct((NWORDS, B), jnp.float32),
        scratch_shapes=[pltpu.VMEM((B, HID), jnp.float32)],
        compiler_params=pltpu.CompilerParams(
            vmem_limit_bytes=100 * 1024 * 1024,
        ),
    )(feat, W1, b1.reshape(1, HID), W2, b2.reshape(NWORDS, 1))
    return outT.T


def kernel(x, emb_table, W1, b1, W2, b2):
    idx = x.astype(jnp.int32).reshape(_NW, _NCH, _CHUNK)
    rows = _sc_gather(idx, emb_table)
    feat = rows.reshape(B, NHIST * EMB)
    return _mlp(feat, W1, b1, W2, b2)
